# Initial kernel scaffold; baseline (speedup 1.0000x reference)
import jax
import jax.numpy as jnp
from jax.experimental import pallas as pl


def kernel(snapshot_sequence, edge_index, conv1_W, conv1_b, conv2_W, conv2_b, conv3_W, conv3_b, lstm_W_ih0, lstm_W_hh0, lstm_b_ih0, lstm_b_hh0, lstm_W_ih1, lstm_W_hh1, lstm_b_ih1, lstm_b_hh1, head_W1, head_b1, head_W2, head_b2):
    raise NotImplementedError("write your pallas kernel here")


# speedup vs baseline: 1.0002x; 1.0002x over previous
import jax
import jax.numpy as jnp
from jax.experimental import pallas as pl


def _gcn_conv(x, src, dst, W, b):
    n = x.shape[0]
    deg = jnp.zeros((n,), dtype=x.dtype).at[dst].add(1.0)
    dis = jnp.where(deg > 0, deg ** -0.5, 0.0)
    norm = dis[src] * dis[dst]
    xw = x @ W.T
    msgs = xw[src] * norm[:, None]
    out = jnp.zeros_like(xw).at[dst].add(msgs)
    return out + b


def _lstm_layer(inputs, W_ih, W_hh, b_ih, b_hh):
    B = inputs.shape[0]
    H = W_hh.shape[1]

    def step(carry, x_t):
        h, c = carry
        gates = x_t @ W_ih.T + h @ W_hh.T + b_ih + b_hh
        i, f, g, o = jnp.split(gates, 4, axis=-1)
        i = jax.nn.sigmoid(i)
        f = jax.nn.sigmoid(f)
        g = jnp.tanh(g)
        o = jax.nn.sigmoid(o)
        c = f * c + i * g
        h = o * jnp.tanh(c)
        return (h, c), h

    h0 = jnp.zeros((B, H), dtype=inputs.dtype)
    c0 = jnp.zeros((B, H), dtype=inputs.dtype)
    _, hs = jax.lax.scan(step, (h0, c0), jnp.swapaxes(inputs, 0, 1))
    return jnp.swapaxes(hs, 0, 1)


def kernel(snapshot_sequence, edge_index, conv1_W, conv1_b, conv2_W, conv2_b, conv3_W, conv3_b, lstm_W_ih0, lstm_W_hh0, lstm_b_ih0, lstm_b_hh0, lstm_W_ih1, lstm_W_hh1, lstm_b_ih1, lstm_b_hh1, head_W1, head_b1, head_W2, head_b2):
    B, T, N, F = snapshot_sequence.shape
    sl = jnp.arange(N, dtype=edge_index.dtype)
    src = jnp.concatenate([edge_index[0], sl])
    dst = jnp.concatenate([edge_index[1], sl])

    def enc(x):
        h = jax.nn.silu(_gcn_conv(x, src, dst, conv1_W, conv1_b))
        h = jax.nn.silu(_gcn_conv(h, src, dst, conv2_W, conv2_b))
        h = jax.nn.silu(_gcn_conv(h, src, dst, conv3_W, conv3_b))
        return jnp.mean(h, axis=0)

    flat = snapshot_sequence.reshape(B * T, N, F)
    emb = jax.vmap(enc)(flat).reshape(B, T, -1)
    h = _lstm_layer(emb, lstm_W_ih0, lstm_W_hh0, lstm_b_ih0, lstm_b_hh0)
    h = _lstm_layer(h, lstm_W_ih1, lstm_W_hh1, lstm_b_ih1, lstm_b_hh1)
    final = h[:, -1, :]
    hid = jax.nn.silu(final @ head_W1.T + head_b1)
    return hid @ head_W2.T + head_b2


# speedup vs baseline: 20.1755x; 20.1724x over previous
import functools

import jax
import jax.numpy as jnp
from jax import lax
from jax.experimental import pallas as pl
from jax.experimental.pallas import tpu as pltpu
from jax.experimental.pallas import tpu_sc as plsc

NC = 2
NS = 16
NW = NC * NS
K = 128
NBUF = 2
NIDX = 4
PADROWS = 240


@functools.lru_cache(maxsize=None)
def _sc_degree_kernel(npad, ch):
    mesh = plsc.VectorSubcoreMesh(core_axis_name="c", subcore_axis_name="s")
    rows_w = npad // NS

    def body(idx_h, out_h, idx_v, ones_v, zer_v, acc):
        c = lax.axis_index("c")
        s = lax.axis_index("s")
        wid = s * NC + c

        pltpu.sync_copy(idx_h.at[wid], idx_v)

        def _zfill(i, carry):
            zer_v[pl.ds(i * 16, 16)] = jnp.zeros((16,), jnp.float32)
            return carry

        lax.fori_loop(0, rows_w // 16, _zfill, 0)

        def _ofill(i, carry):
            ones_v[pl.ds(i * 16, 16)] = jnp.ones((16,), jnp.float32)
            return carry

        lax.fori_loop(0, K // 16, _ofill, 0)

        pltpu.sync_copy(zer_v, acc.at[pl.ds(s * rows_w, rows_w)])
        plsc.subcore_barrier()

        def _chunk(j, carry):
            pltpu.sync_copy(ones_v, acc.at[idx_v.at[j, 1]], add=True)
            return carry

        lax.fori_loop(0, ch, _chunk, 0)
        plsc.subcore_barrier()
        pltpu.sync_copy(acc.at[pl.ds(s * rows_w, rows_w)],
                        out_h.at[c, pl.ds(s * rows_w, rows_w)])

    return pl.kernel(
        body,
        out_type=jax.ShapeDtypeStruct((NC, npad), jnp.float32),
        mesh=mesh,
        scratch_types=[
            pltpu.VMEM((ch, 2, K), jnp.int32),
            pltpu.VMEM((K,), jnp.float32),
            pltpu.VMEM((rows_w,), jnp.float32),
            pltpu.VMEM_SHARED((npad,), jnp.float32),
        ],
    )


@functools.lru_cache(maxsize=None)
def _sc_aggregate_kernel(n, npad, ch):
    mesh = plsc.VectorSubcoreMesh(core_axis_name="c", subcore_axis_name="s")
    rows_w = npad // NS
    zr = 32
    assert rows_w % zr == 0
    assert ch % NIDX == 0

    def body(y_h, idx_h, out_h, islot, buf, zer,
             isem0, isem1, isem2, isem3, gsem0, gsem1, ssem0, ssem1, acc):
        c = lax.axis_index("c")
        s = lax.axis_index("s")
        wid = s * NC + c
        isems = (isem0, isem1, isem2, isem3)
        gsems = (gsem0, gsem1)
        ssems = (ssem0, ssem1)

        def _zfill(i, carry):
            for f in range(8):
                zer[i, pl.ds(f * 16, 16)] = jnp.zeros((16,), jnp.float32)
            return carry

        lax.fori_loop(0, zr, _zfill, 0)

        for r in range(rows_w // zr):
            pltpu.sync_copy(zer, acc.at[pl.ds(s * rows_w + r * zr, zr)])
        plsc.subcore_barrier()

        for q in range(NBUF):
            pltpu.async_copy(idx_h.at[wid, q], islot.at[q], isems[q])

        def _step(j4, carry):
            for b4 in range(NIDX):
                j = j4 * NIDX + b4
                b = b4 % NBUF
                q = b4

                @pl.when(j >= NBUF)
                def _wait_prev_scatter():
                    pltpu.make_async_copy(
                        buf.at[b], acc.at[islot.at[q, 1]], ssems[b]
                    ).wait()

                @pl.when(j + NBUF < ch)
                def _prefetch_idx():
                    pltpu.async_copy(idx_h.at[wid, j + NBUF],
                                     islot.at[(q + NBUF) % NIDX],
                                     isems[(q + NBUF) % NIDX])

                pltpu.make_async_copy(idx_h.at[wid, j], islot.at[q],
                                      isems[q]).wait()
                pltpu.async_copy(y_h.at[islot.at[q, 0]], buf.at[b],
                                 gsems[b]).wait()
                pltpu.async_copy(buf.at[b], acc.at[islot.at[q, 1]],
                                 ssems[b], add=True)
            return carry

        lax.fori_loop(0, ch // NIDX, _step, 0)
        for b4 in range(NIDX - NBUF, NIDX):
            b = b4 % NBUF
            pltpu.make_async_copy(
                buf.at[b], acc.at[islot.at[b4, 1]], ssems[b]
            ).wait()
        plsc.subcore_barrier()
        pltpu.sync_copy(acc.at[pl.ds(s * rows_w, rows_w)],
                        out_h.at[c, pl.ds(s * rows_w, rows_w)])

    return pl.kernel(
        body,
        out_type=jax.ShapeDtypeStruct((NC, npad, 128), jnp.float32),
        mesh=mesh,
        scratch_types=[
            pltpu.VMEM((NIDX, 2, K), jnp.int32),
            pltpu.VMEM((NBUF, K, 128), jnp.float32),
            pltpu.VMEM((zr, 128), jnp.float32),
            pltpu.SemaphoreType.DMA,
            pltpu.SemaphoreType.DMA,
            pltpu.SemaphoreType.DMA,
            pltpu.SemaphoreType.DMA,
            pltpu.SemaphoreType.DMA,
            pltpu.SemaphoreType.DMA,
            pltpu.SemaphoreType.DMA,
            pltpu.SemaphoreType.DMA,
            pltpu.VMEM_SHARED((npad, 128), jnp.float32),
        ],
    )


def _tc_first_layer(x_all, t, wt, dis, n, d):
    def body(x_ref, w_ref, dis_ref, y_ref):
        x = x_ref[0]
        y = jnp.dot(x, w_ref[...], preferred_element_type=jnp.float32)
        y_ref[...] = y * dis_ref[...]

    f = x_all.shape[-1]
    return pl.pallas_call(
        body,
        grid=(1,),
        out_shape=jax.ShapeDtypeStruct((n, d), jnp.float32),
        in_specs=[
            pl.BlockSpec((1, n, f), lambda i: (t, 0, 0)),
            pl.BlockSpec((f, d), lambda i: (0, 0)),
            pl.BlockSpec((n, 1), lambda i: (0, 0)),
        ],
        out_specs=pl.BlockSpec((n, d), lambda i: (0, 0)),
    )(x_all, wt, dis)


def _tc_mid_layer(parts, wt, dis, b_prev, n, d):
    def body(a0_ref, a1_ref, w_ref, dis_ref, b_ref, y_ref):
        h = a0_ref[0] + a1_ref[0]
        h = h * dis_ref[...] + b_ref[...]
        h = h * jax.nn.sigmoid(h)
        y = jnp.dot(h, w_ref[...], preferred_element_type=jnp.float32)
        y_ref[...] = y * dis_ref[...]

    return pl.pallas_call(
        body,
        grid=(1,),
        out_shape=jax.ShapeDtypeStruct((n, d), jnp.float32),
        in_specs=[
            pl.BlockSpec((1, n, d), lambda i: (0, 0, 0)),
            pl.BlockSpec((1, n, d), lambda i: (1, 0, 0)),
            pl.BlockSpec((d, d), lambda i: (0, 0)),
            pl.BlockSpec((n, 1), lambda i: (0, 0)),
            pl.BlockSpec((1, d), lambda i: (0, 0)),
        ],
        out_specs=pl.BlockSpec((n, d), lambda i: (0, 0)),
    )(parts, parts, wt, dis, b_prev)


def _tc_final_layer(parts, dis, b3, n, d):
    def body(a0_ref, a1_ref, dis_ref, b_ref, out_ref):
        h = a0_ref[0] + a1_ref[0]
        h = h * dis_ref[...] + b_ref[...]
        h = h * jax.nn.sigmoid(h)
        out_ref[...] = jnp.sum(h, axis=0, keepdims=True) * (1.0 / n)

    return pl.pallas_call(
        body,
        grid=(1,),
        out_shape=jax.ShapeDtypeStruct((1, d), jnp.float32),
        in_specs=[
            pl.BlockSpec((1, n, d), lambda i: (0, 0, 0)),
            pl.BlockSpec((1, n, d), lambda i: (1, 0, 0)),
            pl.BlockSpec((n, 1), lambda i: (0, 0)),
            pl.BlockSpec((1, d), lambda i: (0, 0)),
        ],
        out_specs=pl.BlockSpec((1, d), lambda i: (0, 0)),
    )(parts, parts, dis, b3)


def _tc_temporal(emb, wih0, whh0, b0, wih1, whh1, b1, hw1, hb1, hw2, hb2):
    B, T, D = emb.shape
    H = D

    def body(e_ref, wih0_ref, whh0_ref, b0_ref, wih1_ref, whh1_ref, b1_ref,
             hw1_ref, hb1_ref, hw2_ref, hb2_ref, out_ref):
        x = e_ref[...]
        h0 = jnp.zeros((B, H), jnp.float32)
        c0 = jnp.zeros((B, H), jnp.float32)
        h1 = jnp.zeros((B, H), jnp.float32)
        c1 = jnp.zeros((B, H), jnp.float32)

        def step(xt, h, c, wih_ref, whh_ref, b_ref):
            gates = (jnp.dot(xt, wih_ref[...], preferred_element_type=jnp.float32)
                     + jnp.dot(h, whh_ref[...], preferred_element_type=jnp.float32)
                     + b_ref[...])
            i = jax.nn.sigmoid(gates[:, 0:H])
            f = jax.nn.sigmoid(gates[:, H:2 * H])
            g = jnp.tanh(gates[:, 2 * H:3 * H])
            o = jax.nn.sigmoid(gates[:, 3 * H:4 * H])
            c = f * c + i * g
            h = o * jnp.tanh(c)
            return h, c

        for t in range(T):
            xt = x[:, t, :]
            h0, c0 = step(xt, h0, c0, wih0_ref, whh0_ref, b0_ref)
            h1, c1 = step(h0, h1, c1, wih1_ref, whh1_ref, b1_ref)

        hid = jnp.dot(h1, hw1_ref[...], preferred_element_type=jnp.float32) + hb1_ref[...]
        hid = hid * jax.nn.sigmoid(hid)
        out_ref[...] = jnp.dot(hid, hw2_ref[...],
                               preferred_element_type=jnp.float32) + hb2_ref[...]

    nout = hw2.shape[1]
    full = lambda shape: pl.BlockSpec(shape, lambda i: (0,) * len(shape))
    return pl.pallas_call(
        body,
        grid=(1,),
        out_shape=jax.ShapeDtypeStruct((B, nout), jnp.float32),
        in_specs=[
            full((B, T, D)),
            full(wih0.shape), full(whh0.shape), full(b0.shape),
            full(wih1.shape), full(whh1.shape), full(b1.shape),
            full(hw1.shape), full(hb1.shape), full(hw2.shape), full(hb2.shape),
        ],
        out_specs=full((B, nout)),
    )(emb, wih0, whh0, b0, wih1, whh1, b1, hw1, hb1, hw2, hb2)


def kernel(snapshot_sequence, edge_index, conv1_W, conv1_b, conv2_W, conv2_b,
           conv3_W, conv3_b, lstm_W_ih0, lstm_W_hh0, lstm_b_ih0, lstm_b_hh0,
           lstm_W_ih1, lstm_W_hh1, lstm_b_ih1, lstm_b_hh1, head_W1, head_b1,
           head_W2, head_b2):
    B, T, N, F = snapshot_sequence.shape
    D = conv1_W.shape[0]
    E = edge_index.shape[1]
    BT = B * T

    npad = N + PADROWS
    assert npad % NS == 0
    edges = E + N
    ch = -(-edges // (NW * K))
    ch += (-ch) % NIDX
    e_pad = NW * ch * K
    pad = e_pad - edges

    idt = edge_index.dtype
    sl = jnp.arange(N, dtype=idt)
    pad_src = jnp.arange(pad, dtype=idt) % N
    pad_dst = N + jnp.arange(pad, dtype=idt) % PADROWS
    src_p = jnp.concatenate([edge_index[0], sl, pad_src]).reshape(NW, ch, K)
    dst_p = jnp.concatenate([edge_index[1], sl, pad_dst]).reshape(NW, ch, K)
    idx_p = jnp.stack([src_p, dst_p], axis=2)

    degp = _sc_degree_kernel(npad, ch)(idx_p)
    deg = degp[0] + degp[1]
    dis = jnp.where(deg > 0, lax.rsqrt(deg), 0.0)[:, None]

    w1t = conv1_W.T
    w2t = conv2_W.T
    w3t = conv3_W.T
    b1r = conv1_b[None, :]
    b2r = conv2_b[None, :]
    b3r = conv3_b[None, :]

    x_all = snapshot_sequence.reshape(BT, N, F)
    agg = _sc_aggregate_kernel(N, npad, ch)

    embs = []
    for t in range(BT):
        y1 = _tc_first_layer(x_all, t, w1t, dis, N, D)
        p1 = agg(y1, idx_p)
        y2 = _tc_mid_layer(p1, w2t, dis, b1r, N, D)
        p2 = agg(y2, idx_p)
        y3 = _tc_mid_layer(p2, w3t, dis, b2r, N, D)
        p3 = agg(y3, idx_p)
        embs.append(_tc_final_layer(p3, dis, b3r, N, D))

    emb = jnp.concatenate(embs, axis=0).reshape(B, T, D)

    out = _tc_temporal(
        emb,
        lstm_W_ih0.T, lstm_W_hh0.T, (lstm_b_ih0 + lstm_b_hh0)[None, :],
        lstm_W_ih1.T, lstm_W_hh1.T, (lstm_b_ih1 + lstm_b_hh1)[None, :],
        head_W1.T, head_b1[None, :], head_W2.T, head_b2[None, :])
    return out


# speedup vs baseline: 24.3768x; 1.2082x over previous
import functools

import jax
import jax.numpy as jnp
from jax import lax
from jax.experimental import pallas as pl
from jax.experimental.pallas import tpu as pltpu
from jax.experimental.pallas import tpu_sc as plsc

NC = 2
NS = 16
NW = NC * NS
K = 128
NBUF = 2
NIDX = 4
PADROWS = 240


@functools.lru_cache(maxsize=None)
def _sc_degree_kernel(npad, ch):
    mesh = plsc.VectorSubcoreMesh(core_axis_name="c", subcore_axis_name="s")
    rows_w = npad // NS

    def body(idx_h, out_h, idx_v, ones_v, zer_v, acc):
        c = lax.axis_index("c")
        s = lax.axis_index("s")
        wid = s * NC + c

        pltpu.sync_copy(idx_h.at[wid], idx_v)

        def _zfill(i, carry):
            zer_v[pl.ds(i * 16, 16)] = jnp.zeros((16,), jnp.float32)
            return carry

        lax.fori_loop(0, rows_w // 16, _zfill, 0)

        def _ofill(i, carry):
            ones_v[pl.ds(i * 16, 16)] = jnp.ones((16,), jnp.float32)
            return carry

        lax.fori_loop(0, K // 16, _ofill, 0)

        pltpu.sync_copy(zer_v, acc.at[pl.ds(s * rows_w, rows_w)])
        plsc.subcore_barrier()

        def _chunk(j, carry):
            pltpu.sync_copy(ones_v, acc.at[idx_v.at[j, 1]], add=True)
            return carry

        lax.fori_loop(0, ch, _chunk, 0)
        plsc.subcore_barrier()
        pltpu.sync_copy(acc.at[pl.ds(s * rows_w, rows_w)],
                        out_h.at[c, pl.ds(s * rows_w, rows_w)])

    return pl.kernel(
        body,
        out_type=jax.ShapeDtypeStruct((NC, npad), jnp.float32),
        mesh=mesh,
        scratch_types=[
            pltpu.VMEM((ch, 2, K), jnp.int32),
            pltpu.VMEM((K,), jnp.float32),
            pltpu.VMEM((rows_w,), jnp.float32),
            pltpu.VMEM_SHARED((npad,), jnp.float32),
        ],
    )


@functools.lru_cache(maxsize=None)
def _sc_aggregate_kernel(n, npad, ch):
    mesh = plsc.VectorSubcoreMesh(core_axis_name="c", subcore_axis_name="s")
    rows_w = npad // NS
    zr = 32
    assert rows_w % zr == 0
    assert ch % NIDX == 0

    def body(y_h, idx_h, out_h, islot, buf, zer,
             isem0, isem1, isem2, isem3, gsem0, gsem1, ssem0, ssem1, acc):
        c = lax.axis_index("c")
        s = lax.axis_index("s")
        wid = s * NC + c
        isems = (isem0, isem1, isem2, isem3)
        gsems = (gsem0, gsem1)
        ssems = (ssem0, ssem1)

        def _zfill(i, carry):
            for f in range(8):
                zer[i, pl.ds(f * 16, 16)] = jnp.zeros((16,), jnp.float32)
            return carry

        lax.fori_loop(0, zr, _zfill, 0)

        for r in range(rows_w // zr):
            pltpu.async_copy(zer, acc.at[pl.ds(s * rows_w + r * zr, zr)],
                             gsems[0])
        for r in range(rows_w // zr):
            pltpu.make_async_copy(zer, acc.at[pl.ds(s * rows_w, zr)],
                                  gsems[0]).wait()
        plsc.subcore_barrier()

        for q in range(NIDX - 1):
            pltpu.async_copy(idx_h.at[wid, q], islot.at[q], isems[q])
        pltpu.make_async_copy(idx_h.at[wid, 0], islot.at[0], isems[0]).wait()
        pltpu.async_copy(y_h.at[islot.at[0, 0]], buf.at[0], gsems[0])

        def _step(j4, carry):
            for b4 in range(NIDX):
                j = j4 * NIDX + b4
                b = b4 % NBUF
                q = b4

                @pl.when(j > 0)
                def _wait_prev_scatter():
                    pltpu.make_async_copy(
                        buf.at[(b + 1) % NBUF], acc.at[islot.at[q, 1]],
                        ssems[(b + 1) % NBUF]
                    ).wait()

                @pl.when(j + 1 < ch)
                def _issue_next_gather():
                    pltpu.make_async_copy(idx_h.at[wid, j + 1],
                                          islot.at[(q + 1) % NIDX],
                                          isems[(q + 1) % NIDX]).wait()
                    pltpu.async_copy(y_h.at[islot.at[(q + 1) % NIDX, 0]],
                                     buf.at[(b + 1) % NBUF],
                                     gsems[(b + 1) % NBUF])

                @pl.when(j + NIDX - 1 < ch)
                def _prefetch_idx():
                    pltpu.async_copy(idx_h.at[wid, j + NIDX - 1],
                                     islot.at[(q + NIDX - 1) % NIDX],
                                     isems[(q + NIDX - 1) % NIDX])

                pltpu.make_async_copy(y_h.at[islot.at[q, 0]], buf.at[b],
                                      gsems[b]).wait()
                pltpu.async_copy(buf.at[b], acc.at[islot.at[q, 1]],
                                 ssems[b], add=True)
            return carry

        lax.fori_loop(0, ch // NIDX, _step, 0)
        pltpu.make_async_copy(
            buf.at[(ch - 1) % NBUF], acc.at[islot.at[(ch - 1) % NIDX, 1]],
            ssems[(ch - 1) % NBUF]
        ).wait()
        plsc.subcore_barrier()
        pltpu.sync_copy(acc.at[pl.ds(s * rows_w, rows_w)],
                        out_h.at[c, pl.ds(s * rows_w, rows_w)])

    return pl.kernel(
        body,
        out_type=jax.ShapeDtypeStruct((NC, npad, 128), jnp.float32),
        mesh=mesh,
        scratch_types=[
            pltpu.VMEM((NIDX, 2, K), jnp.int32),
            pltpu.VMEM((NBUF, K, 128), jnp.float32),
            pltpu.VMEM((zr, 128), jnp.float32),
            pltpu.SemaphoreType.DMA,
            pltpu.SemaphoreType.DMA,
            pltpu.SemaphoreType.DMA,
            pltpu.SemaphoreType.DMA,
            pltpu.SemaphoreType.DMA,
            pltpu.SemaphoreType.DMA,
            pltpu.SemaphoreType.DMA,
            pltpu.SemaphoreType.DMA,
            pltpu.VMEM_SHARED((npad, 128), jnp.float32),
        ],
    )


def _tc_first_layer(x_all, t, wt, dis, n, d):
    def body(x_ref, w_ref, dis_ref, y_ref):
        x = x_ref[0]
        y = jnp.dot(x, w_ref[...], preferred_element_type=jnp.float32)
        y_ref[...] = y * dis_ref[...]

    f = x_all.shape[-1]
    return pl.pallas_call(
        body,
        grid=(1,),
        out_shape=jax.ShapeDtypeStruct((n, d), jnp.float32),
        in_specs=[
            pl.BlockSpec((1, n, f), lambda i: (t, 0, 0)),
            pl.BlockSpec((f, d), lambda i: (0, 0)),
            pl.BlockSpec((n, 1), lambda i: (0, 0)),
        ],
        out_specs=pl.BlockSpec((n, d), lambda i: (0, 0)),
    )(x_all, wt, dis)


def _tc_mid_layer(parts, wt, dis, b_prev, n, d):
    def body(a0_ref, a1_ref, w_ref, dis_ref, b_ref, y_ref):
        h = a0_ref[0] + a1_ref[0]
        h = h * dis_ref[...] + b_ref[...]
        h = h * jax.nn.sigmoid(h)
        y = jnp.dot(h, w_ref[...], preferred_element_type=jnp.float32)
        y_ref[...] = y * dis_ref[...]

    return pl.pallas_call(
        body,
        grid=(1,),
        out_shape=jax.ShapeDtypeStruct((n, d), jnp.float32),
        in_specs=[
            pl.BlockSpec((1, n, d), lambda i: (0, 0, 0)),
            pl.BlockSpec((1, n, d), lambda i: (1, 0, 0)),
            pl.BlockSpec((d, d), lambda i: (0, 0)),
            pl.BlockSpec((n, 1), lambda i: (0, 0)),
            pl.BlockSpec((1, d), lambda i: (0, 0)),
        ],
        out_specs=pl.BlockSpec((n, d), lambda i: (0, 0)),
    )(parts, parts, wt, dis, b_prev)


def _tc_final_layer(parts, dis, b3, n, d):
    def body(a0_ref, a1_ref, dis_ref, b_ref, out_ref):
        h = a0_ref[0] + a1_ref[0]
        h = h * dis_ref[...] + b_ref[...]
        h = h * jax.nn.sigmoid(h)
        out_ref[...] = jnp.sum(h, axis=0, keepdims=True) * (1.0 / n)

    return pl.pallas_call(
        body,
        grid=(1,),
        out_shape=jax.ShapeDtypeStruct((1, d), jnp.float32),
        in_specs=[
            pl.BlockSpec((1, n, d), lambda i: (0, 0, 0)),
            pl.BlockSpec((1, n, d), lambda i: (1, 0, 0)),
            pl.BlockSpec((n, 1), lambda i: (0, 0)),
            pl.BlockSpec((1, d), lambda i: (0, 0)),
        ],
        out_specs=pl.BlockSpec((1, d), lambda i: (0, 0)),
    )(parts, parts, dis, b3)


def _tc_temporal(emb, wih0, whh0, b0, wih1, whh1, b1, hw1, hb1, hw2, hb2):
    B, T, D = emb.shape
    H = D

    def body(e_ref, wih0_ref, whh0_ref, b0_ref, wih1_ref, whh1_ref, b1_ref,
             hw1_ref, hb1_ref, hw2_ref, hb2_ref, out_ref):
        x = e_ref[...]
        h0 = jnp.zeros((B, H), jnp.float32)
        c0 = jnp.zeros((B, H), jnp.float32)
        h1 = jnp.zeros((B, H), jnp.float32)
        c1 = jnp.zeros((B, H), jnp.float32)

        def step(xt, h, c, wih_ref, whh_ref, b_ref):
            gates = (jnp.dot(xt, wih_ref[...], preferred_element_type=jnp.float32)
                     + jnp.dot(h, whh_ref[...], preferred_element_type=jnp.float32)
                     + b_ref[...])
            i = jax.nn.sigmoid(gates[:, 0:H])
            f = jax.nn.sigmoid(gates[:, H:2 * H])
            g = jnp.tanh(gates[:, 2 * H:3 * H])
            o = jax.nn.sigmoid(gates[:, 3 * H:4 * H])
            c = f * c + i * g
            h = o * jnp.tanh(c)
            return h, c

        for t in range(T):
            xt = x[:, t, :]
            h0, c0 = step(xt, h0, c0, wih0_ref, whh0_ref, b0_ref)
            h1, c1 = step(h0, h1, c1, wih1_ref, whh1_ref, b1_ref)

        hid = jnp.dot(h1, hw1_ref[...], preferred_element_type=jnp.float32) + hb1_ref[...]
        hid = hid * jax.nn.sigmoid(hid)
        out_ref[...] = jnp.dot(hid, hw2_ref[...],
                               preferred_element_type=jnp.float32) + hb2_ref[...]

    nout = hw2.shape[1]
    full = lambda shape: pl.BlockSpec(shape, lambda i: (0,) * len(shape))
    return pl.pallas_call(
        body,
        grid=(1,),
        out_shape=jax.ShapeDtypeStruct((B, nout), jnp.float32),
        in_specs=[
            full((B, T, D)),
            full(wih0.shape), full(whh0.shape), full(b0.shape),
            full(wih1.shape), full(whh1.shape), full(b1.shape),
            full(hw1.shape), full(hb1.shape), full(hw2.shape), full(hb2.shape),
        ],
        out_specs=full((B, nout)),
    )(emb, wih0, whh0, b0, wih1, whh1, b1, hw1, hb1, hw2, hb2)


def kernel(snapshot_sequence, edge_index, conv1_W, conv1_b, conv2_W, conv2_b,
           conv3_W, conv3_b, lstm_W_ih0, lstm_W_hh0, lstm_b_ih0, lstm_b_hh0,
           lstm_W_ih1, lstm_W_hh1, lstm_b_ih1, lstm_b_hh1, head_W1, head_b1,
           head_W2, head_b2):
    B, T, N, F = snapshot_sequence.shape
    D = conv1_W.shape[0]
    E = edge_index.shape[1]
    BT = B * T

    npad = N + PADROWS
    assert npad % NS == 0
    edges = E + N
    ch = -(-edges // (NW * K))
    ch += (-ch) % NIDX
    e_pad = NW * ch * K
    pad = e_pad - edges

    idt = edge_index.dtype
    sl = jnp.arange(N, dtype=idt)
    pad_src = jnp.arange(pad, dtype=idt) % N
    pad_dst = N + jnp.arange(pad, dtype=idt) % PADROWS
    src_p = jnp.concatenate([edge_index[0], sl, pad_src]).reshape(NW, ch, K)
    dst_p = jnp.concatenate([edge_index[1], sl, pad_dst]).reshape(NW, ch, K)
    idx_p = jnp.stack([src_p, dst_p], axis=2)

    degp = _sc_degree_kernel(npad, ch)(idx_p)
    deg = degp[0] + degp[1]
    dis = jnp.where(deg > 0, lax.rsqrt(deg), 0.0)[:, None]

    w1t = conv1_W.T
    w2t = conv2_W.T
    w3t = conv3_W.T
    b1r = conv1_b[None, :]
    b2r = conv2_b[None, :]
    b3r = conv3_b[None, :]

    x_all = snapshot_sequence.reshape(BT, N, F)
    agg = _sc_aggregate_kernel(N, npad, ch)

    embs = []
    for t in range(BT):
        y1 = _tc_first_layer(x_all, t, w1t, dis, N, D)
        p1 = agg(y1, idx_p)
        y2 = _tc_mid_layer(p1, w2t, dis, b1r, N, D)
        p2 = agg(y2, idx_p)
        y3 = _tc_mid_layer(p2, w3t, dis, b2r, N, D)
        p3 = agg(y3, idx_p)
        embs.append(_tc_final_layer(p3, dis, b3r, N, D))

    emb = jnp.concatenate(embs, axis=0).reshape(B, T, D)

    out = _tc_temporal(
        emb,
        lstm_W_ih0.T, lstm_W_hh0.T, (lstm_b_ih0 + lstm_b_hh0)[None, :],
        lstm_W_ih1.T, lstm_W_hh1.T, (lstm_b_ih1 + lstm_b_hh1)[None, :],
        head_W1.T, head_b1[None, :], head_W2.T, head_b2[None, :])
    return out


# speedup vs baseline: 25.1181x; 1.0304x over previous
import functools

import jax
import jax.numpy as jnp
from jax import lax
from jax.experimental import pallas as pl
from jax.experimental.pallas import tpu as pltpu
from jax.experimental.pallas import tpu_sc as plsc

NC = 2
NS = 16
NW = NC * NS
K = 120
NBUF = 2
NIDX = 4
PADROWS = 240


@functools.lru_cache(maxsize=None)
def _sc_degree_kernel(npad, ch):
    mesh = plsc.VectorSubcoreMesh(core_axis_name="c", subcore_axis_name="s")
    rows_w = npad // NS

    def body(idx_h, out_h, idx_v, ones_v, zer_v, acc):
        c = lax.axis_index("c")
        s = lax.axis_index("s")
        wid = s * NC + c

        pltpu.sync_copy(idx_h.at[wid], idx_v)

        def _zfill(i, carry):
            zer_v[pl.ds(i * 16, 16)] = jnp.zeros((16,), jnp.float32)
            return carry

        lax.fori_loop(0, rows_w // 16, _zfill, 0)

        def _ofill(i, carry):
            ones_v[pl.ds(i * 16, 16)] = jnp.ones((16,), jnp.float32)
            return carry

        lax.fori_loop(0, K // 16, _ofill, 0)

        pltpu.sync_copy(zer_v, acc.at[pl.ds(s * rows_w, rows_w)])
        plsc.subcore_barrier()

        def _chunk(j, carry):
            pltpu.sync_copy(ones_v, acc.at[idx_v.at[j, 1]], add=True)
            return carry

        lax.fori_loop(0, ch, _chunk, 0)
        plsc.subcore_barrier()
        pltpu.sync_copy(acc.at[pl.ds(s * rows_w, rows_w)],
                        out_h.at[c, pl.ds(s * rows_w, rows_w)])

    return pl.kernel(
        body,
        out_type=jax.ShapeDtypeStruct((NC, npad), jnp.float32),
        mesh=mesh,
        scratch_types=[
            pltpu.VMEM((ch, 2, K), jnp.int32),
            pltpu.VMEM((K,), jnp.float32),
            pltpu.VMEM((rows_w,), jnp.float32),
            pltpu.VMEM_SHARED((npad,), jnp.float32),
        ],
    )


@functools.lru_cache(maxsize=None)
def _sc_aggregate_kernel(n, npad, ch):
    mesh = plsc.VectorSubcoreMesh(core_axis_name="c", subcore_axis_name="s")
    rows_w = npad // NS
    zr = 32
    assert rows_w % zr == 0
    assert ch % NIDX == 0

    def body(y_h, idx_h, out_h, islot, buf, zer,
             isem0, isem1, isem2, isem3, gsem0, gsem1, ssem0, ssem1, acc):
        c = lax.axis_index("c")
        s = lax.axis_index("s")
        wid = s * NC + c
        isems = (isem0, isem1, isem2, isem3)
        gsems = (gsem0, gsem1)
        ssems = (ssem0, ssem1)

        def _zfill(i, carry):
            for f in range(8):
                zer[i, pl.ds(f * 16, 16)] = jnp.zeros((16,), jnp.float32)
            return carry

        lax.fori_loop(0, zr, _zfill, 0)

        for r in range(rows_w // zr):
            pltpu.async_copy(zer, acc.at[pl.ds(s * rows_w + r * zr, zr)],
                             ssems[0])
        for q in range(NIDX - 1):
            pltpu.async_copy(idx_h.at[wid, q], islot.at[q], isems[q])
        pltpu.make_async_copy(idx_h.at[wid, 0], islot.at[0], isems[0]).wait()
        pltpu.async_copy(y_h.at[islot.at[0, 0]], buf.at[0], gsems[0])
        for r in range(rows_w // zr):
            pltpu.make_async_copy(zer, acc.at[pl.ds(s * rows_w, zr)],
                                  ssems[0]).wait()
        plsc.subcore_barrier()

        def _step(j4, carry):
            for b4 in range(NIDX):
                j = j4 * NIDX + b4
                b = b4 % NBUF
                q = b4

                @pl.when(j > 0)
                def _wait_prev_scatter():
                    pltpu.make_async_copy(
                        buf.at[(b + 1) % NBUF], acc.at[islot.at[q, 1]],
                        ssems[(b + 1) % NBUF]
                    ).wait()

                @pl.when(j + 1 < ch)
                def _issue_next_gather():
                    pltpu.make_async_copy(idx_h.at[wid, j + 1],
                                          islot.at[(q + 1) % NIDX],
                                          isems[(q + 1) % NIDX]).wait()
                    pltpu.async_copy(y_h.at[islot.at[(q + 1) % NIDX, 0]],
                                     buf.at[(b + 1) % NBUF],
                                     gsems[(b + 1) % NBUF])

                @pl.when(j + NIDX - 1 < ch)
                def _prefetch_idx():
                    pltpu.async_copy(idx_h.at[wid, j + NIDX - 1],
                                     islot.at[(q + NIDX - 1) % NIDX],
                                     isems[(q + NIDX - 1) % NIDX])

                pltpu.make_async_copy(y_h.at[islot.at[q, 0]], buf.at[b],
                                      gsems[b]).wait()
                pltpu.async_copy(buf.at[b], acc.at[islot.at[q, 1]],
                                 ssems[b], add=True)
            return carry

        lax.fori_loop(0, ch // NIDX, _step, 0)
        pltpu.make_async_copy(
            buf.at[(ch - 1) % NBUF], acc.at[islot.at[(ch - 1) % NIDX, 1]],
            ssems[(ch - 1) % NBUF]
        ).wait()
        plsc.subcore_barrier()
        pltpu.sync_copy(acc.at[pl.ds(s * rows_w, rows_w)],
                        out_h.at[c, pl.ds(s * rows_w, rows_w)])

    return pl.kernel(
        body,
        out_type=jax.ShapeDtypeStruct((NC, npad, 128), jnp.float32),
        mesh=mesh,
        scratch_types=[
            pltpu.VMEM((NIDX, 2, K), jnp.int32),
            pltpu.VMEM((NBUF, K, 128), jnp.float32),
            pltpu.VMEM((zr, 128), jnp.float32),
            pltpu.SemaphoreType.DMA,
            pltpu.SemaphoreType.DMA,
            pltpu.SemaphoreType.DMA,
            pltpu.SemaphoreType.DMA,
            pltpu.SemaphoreType.DMA,
            pltpu.SemaphoreType.DMA,
            pltpu.SemaphoreType.DMA,
            pltpu.SemaphoreType.DMA,
            pltpu.VMEM_SHARED((npad, 128), jnp.float32),
        ],
    )


def _tc_first_layer(x_all, t, wt, dis, n, d):
    def body(x_ref, w_ref, dis_ref, y_ref):
        x = x_ref[0]
        y = jnp.dot(x, w_ref[...], preferred_element_type=jnp.float32)
        y_ref[...] = y * dis_ref[...]

    f = x_all.shape[-1]
    return pl.pallas_call(
        body,
        grid=(1,),
        out_shape=jax.ShapeDtypeStruct((n, d), jnp.float32),
        in_specs=[
            pl.BlockSpec((1, n, f), lambda i: (t, 0, 0)),
            pl.BlockSpec((f, d), lambda i: (0, 0)),
            pl.BlockSpec((n, 1), lambda i: (0, 0)),
        ],
        out_specs=pl.BlockSpec((n, d), lambda i: (0, 0)),
    )(x_all, wt, dis)


def _tc_mid_layer(parts, wt, dis, b_prev, n, d):
    def body(a0_ref, a1_ref, w_ref, dis_ref, b_ref, y_ref):
        h = a0_ref[0] + a1_ref[0]
        h = h * dis_ref[...] + b_ref[...]
        h = h * jax.nn.sigmoid(h)
        y = jnp.dot(h, w_ref[...], preferred_element_type=jnp.float32)
        y_ref[...] = y * dis_ref[...]

    return pl.pallas_call(
        body,
        grid=(1,),
        out_shape=jax.ShapeDtypeStruct((n, d), jnp.float32),
        in_specs=[
            pl.BlockSpec((1, n, d), lambda i: (0, 0, 0)),
            pl.BlockSpec((1, n, d), lambda i: (1, 0, 0)),
            pl.BlockSpec((d, d), lambda i: (0, 0)),
            pl.BlockSpec((n, 1), lambda i: (0, 0)),
            pl.BlockSpec((1, d), lambda i: (0, 0)),
        ],
        out_specs=pl.BlockSpec((n, d), lambda i: (0, 0)),
    )(parts, parts, wt, dis, b_prev)


def _tc_final_layer(parts, dis, b3, n, d):
    def body(a0_ref, a1_ref, dis_ref, b_ref, out_ref):
        h = a0_ref[0] + a1_ref[0]
        h = h * dis_ref[...] + b_ref[...]
        h = h * jax.nn.sigmoid(h)
        out_ref[...] = jnp.sum(h, axis=0, keepdims=True) * (1.0 / n)

    return pl.pallas_call(
        body,
        grid=(1,),
        out_shape=jax.ShapeDtypeStruct((1, d), jnp.float32),
        in_specs=[
            pl.BlockSpec((1, n, d), lambda i: (0, 0, 0)),
            pl.BlockSpec((1, n, d), lambda i: (1, 0, 0)),
            pl.BlockSpec((n, 1), lambda i: (0, 0)),
            pl.BlockSpec((1, d), lambda i: (0, 0)),
        ],
        out_specs=pl.BlockSpec((1, d), lambda i: (0, 0)),
    )(parts, parts, dis, b3)


def _tc_temporal(emb, wih0, whh0, b0, wih1, whh1, b1, hw1, hb1, hw2, hb2):
    B, T, D = emb.shape
    H = D

    def body(e_ref, wih0_ref, whh0_ref, b0_ref, wih1_ref, whh1_ref, b1_ref,
             hw1_ref, hb1_ref, hw2_ref, hb2_ref, out_ref):
        x = e_ref[...]
        h0 = jnp.zeros((B, H), jnp.float32)
        c0 = jnp.zeros((B, H), jnp.float32)
        h1 = jnp.zeros((B, H), jnp.float32)
        c1 = jnp.zeros((B, H), jnp.float32)

        def step(xt, h, c, wih_ref, whh_ref, b_ref):
            gates = (jnp.dot(xt, wih_ref[...], preferred_element_type=jnp.float32)
                     + jnp.dot(h, whh_ref[...], preferred_element_type=jnp.float32)
                     + b_ref[...])
            i = jax.nn.sigmoid(gates[:, 0:H])
            f = jax.nn.sigmoid(gates[:, H:2 * H])
            g = jnp.tanh(gates[:, 2 * H:3 * H])
            o = jax.nn.sigmoid(gates[:, 3 * H:4 * H])
            c = f * c + i * g
            h = o * jnp.tanh(c)
            return h, c

        for t in range(T):
            xt = x[:, t, :]
            h0, c0 = step(xt, h0, c0, wih0_ref, whh0_ref, b0_ref)
            h1, c1 = step(h0, h1, c1, wih1_ref, whh1_ref, b1_ref)

        hid = jnp.dot(h1, hw1_ref[...], preferred_element_type=jnp.float32) + hb1_ref[...]
        hid = hid * jax.nn.sigmoid(hid)
        out_ref[...] = jnp.dot(hid, hw2_ref[...],
                               preferred_element_type=jnp.float32) + hb2_ref[...]

    nout = hw2.shape[1]
    full = lambda shape: pl.BlockSpec(shape, lambda i: (0,) * len(shape))
    return pl.pallas_call(
        body,
        grid=(1,),
        out_shape=jax.ShapeDtypeStruct((B, nout), jnp.float32),
        in_specs=[
            full((B, T, D)),
            full(wih0.shape), full(whh0.shape), full(b0.shape),
            full(wih1.shape), full(whh1.shape), full(b1.shape),
            full(hw1.shape), full(hb1.shape), full(hw2.shape), full(hb2.shape),
        ],
        out_specs=full((B, nout)),
    )(emb, wih0, whh0, b0, wih1, whh1, b1, hw1, hb1, hw2, hb2)


def kernel(snapshot_sequence, edge_index, conv1_W, conv1_b, conv2_W, conv2_b,
           conv3_W, conv3_b, lstm_W_ih0, lstm_W_hh0, lstm_b_ih0, lstm_b_hh0,
           lstm_W_ih1, lstm_W_hh1, lstm_b_ih1, lstm_b_hh1, head_W1, head_b1,
           head_W2, head_b2):
    B, T, N, F = snapshot_sequence.shape
    D = conv1_W.shape[0]
    E = edge_index.shape[1]
    BT = B * T

    npad = N + PADROWS
    assert npad % NS == 0
    edges = E + N
    ch = -(-edges // (NW * K))
    ch += (-ch) % NIDX
    e_pad = NW * ch * K
    pad = e_pad - edges

    idt = edge_index.dtype
    sl = jnp.arange(N, dtype=idt)
    pad_src = jnp.arange(pad, dtype=idt) % N
    pad_dst = N + jnp.arange(pad, dtype=idt) % PADROWS
    src_p = jnp.concatenate([edge_index[0], sl, pad_src]).reshape(NW, ch, K)
    dst_p = jnp.concatenate([edge_index[1], sl, pad_dst]).reshape(NW, ch, K)
    idx_p = jnp.stack([src_p, dst_p], axis=2)

    degp = _sc_degree_kernel(npad, ch)(idx_p)
    deg = degp[0] + degp[1]
    dis = jnp.where(deg > 0, lax.rsqrt(deg), 0.0)[:, None]

    w1t = conv1_W.T
    w2t = conv2_W.T
    w3t = conv3_W.T
    b1r = conv1_b[None, :]
    b2r = conv2_b[None, :]
    b3r = conv3_b[None, :]

    x_all = snapshot_sequence.reshape(BT, N, F)
    agg = _sc_aggregate_kernel(N, npad, ch)

    embs = []
    for t in range(BT):
        y1 = _tc_first_layer(x_all, t, w1t, dis, N, D)
        p1 = agg(y1, idx_p)
        y2 = _tc_mid_layer(p1, w2t, dis, b1r, N, D)
        p2 = agg(y2, idx_p)
        y3 = _tc_mid_layer(p2, w3t, dis, b2r, N, D)
        p3 = agg(y3, idx_p)
        embs.append(_tc_final_layer(p3, dis, b3r, N, D))

    emb = jnp.concatenate(embs, axis=0).reshape(B, T, D)

    out = _tc_temporal(
        emb,
        lstm_W_ih0.T, lstm_W_hh0.T, (lstm_b_ih0 + lstm_b_hh0)[None, :],
        lstm_W_ih1.T, lstm_W_hh1.T, (lstm_b_ih1 + lstm_b_hh1)[None, :],
        head_W1.T, head_b1[None, :], head_W2.T, head_b2[None, :])
    return out


# speedup vs baseline: 25.7425x; 1.0249x over previous
import functools

import jax
import jax.numpy as jnp
from jax import lax
from jax.experimental import pallas as pl
from jax.experimental.pallas import tpu as pltpu
from jax.experimental.pallas import tpu_sc as plsc

NC = 2
NS = 16
NW = NC * NS
K = 72
NBUF = 4
NIDX = 8
PADROWS = 240


@functools.lru_cache(maxsize=None)
def _sc_degree_kernel(npad, ch):
    mesh = plsc.VectorSubcoreMesh(core_axis_name="c", subcore_axis_name="s")
    rows_w = npad // NS

    def body(idx_h, out_h, idx_v, ones_v, zer_v, acc):
        c = lax.axis_index("c")
        s = lax.axis_index("s")
        wid = s * NC + c

        pltpu.sync_copy(idx_h.at[wid], idx_v)

        def _zfill(i, carry):
            zer_v[pl.ds(i * 16, 16)] = jnp.zeros((16,), jnp.float32)
            return carry

        lax.fori_loop(0, rows_w // 16, _zfill, 0)

        def _ofill(i, carry):
            ones_v[pl.ds(i * 16, 16)] = jnp.ones((16,), jnp.float32)
            return carry

        lax.fori_loop(0, K // 16, _ofill, 0)

        pltpu.sync_copy(zer_v, acc.at[pl.ds(s * rows_w, rows_w)])
        plsc.subcore_barrier()

        def _chunk(j, carry):
            pltpu.sync_copy(ones_v, acc.at[idx_v.at[j, 1]], add=True)
            return carry

        lax.fori_loop(0, ch, _chunk, 0)
        plsc.subcore_barrier()
        pltpu.sync_copy(acc.at[pl.ds(s * rows_w, rows_w)],
                        out_h.at[c, pl.ds(s * rows_w, rows_w)])

    return pl.kernel(
        body,
        out_type=jax.ShapeDtypeStruct((NC, npad), jnp.float32),
        mesh=mesh,
        scratch_types=[
            pltpu.VMEM((ch, 2, K), jnp.int32),
            pltpu.VMEM((K,), jnp.float32),
            pltpu.VMEM((rows_w,), jnp.float32),
            pltpu.VMEM_SHARED((npad,), jnp.float32),
        ],
    )


@functools.lru_cache(maxsize=None)
def _sc_aggregate_kernel(n, npad, ch):
    mesh = plsc.VectorSubcoreMesh(core_axis_name="c", subcore_axis_name="s")
    rows_w = npad // NS
    zr = 8
    assert rows_w % zr == 0
    assert ch % NIDX == 0 and ch >= NIDX

    def body(y_h, idx_h, out_h, islot, buf, zer, sems, acc):
        c = lax.axis_index("c")
        s = lax.axis_index("s")
        wid = s * NC + c
        isems = [sems.at[i] for i in range(NIDX)]
        gsems = [sems.at[NIDX + i] for i in range(NBUF)]
        ssems = [sems.at[NIDX + NBUF + i] for i in range(NBUF)]

        def _zfill(i, carry):
            for f in range(8):
                zer[i, pl.ds(f * 16, 16)] = jnp.zeros((16,), jnp.float32)
            return carry

        lax.fori_loop(0, zr, _zfill, 0)

        for r in range(rows_w // zr):
            pltpu.async_copy(zer, acc.at[pl.ds(s * rows_w + r * zr, zr)],
                             ssems[0])
        for q in range(5):
            pltpu.async_copy(idx_h.at[wid, q], islot.at[q], isems[q])
        for j0 in range(2):
            pltpu.make_async_copy(idx_h.at[wid, j0], islot.at[j0],
                                  isems[j0]).wait()
            pltpu.async_copy(y_h.at[islot.at[j0, 0]], buf.at[j0], gsems[j0])
        for r in range(rows_w // zr):
            pltpu.make_async_copy(zer, acc.at[pl.ds(s * rows_w, zr)],
                                  ssems[0]).wait()
        plsc.subcore_barrier()

        def _step(j8, carry):
            for k in range(NIDX):
                j = j8 * NIDX + k
                b = k % NBUF

                @pl.when(j >= 2)
                def _wait_scatter_jm2():
                    pltpu.make_async_copy(
                        buf.at[(b + 2) % NBUF], acc.at[islot.at[k, 1]],
                        ssems[(b + 2) % NBUF]
                    ).wait()

                @pl.when(j + 2 < ch)
                def _issue_gather_jp2():
                    pltpu.make_async_copy(idx_h.at[wid, j + 2],
                                          islot.at[(k + 2) % NIDX],
                                          isems[(k + 2) % NIDX]).wait()
                    pltpu.async_copy(y_h.at[islot.at[(k + 2) % NIDX, 0]],
                                     buf.at[(b + 2) % NBUF],
                                     gsems[(b + 2) % NBUF])

                @pl.when(j + 5 < ch)
                def _prefetch_idx():
                    pltpu.async_copy(idx_h.at[wid, j + 5],
                                     islot.at[(k + 5) % NIDX],
                                     isems[(k + 5) % NIDX])

                pltpu.make_async_copy(y_h.at[islot.at[k, 0]], buf.at[b],
                                      gsems[b]).wait()
                pltpu.async_copy(buf.at[b], acc.at[islot.at[k, 1]],
                                 ssems[b], add=True)
            return carry

        lax.fori_loop(0, ch // NIDX, _step, 0)
        for jt in (ch - 2, ch - 1):
            pltpu.make_async_copy(
                buf.at[jt % NBUF], acc.at[islot.at[jt % NIDX, 1]],
                ssems[jt % NBUF]
            ).wait()
        plsc.subcore_barrier()
        pltpu.sync_copy(acc.at[pl.ds(s * rows_w, rows_w)],
                        out_h.at[c, pl.ds(s * rows_w, rows_w)])

    return pl.kernel(
        body,
        out_type=jax.ShapeDtypeStruct((NC, npad, 128), jnp.float32),
        mesh=mesh,
        scratch_types=[
            pltpu.VMEM((NIDX, 2, K), jnp.int32),
            pltpu.VMEM((NBUF, K, 128), jnp.float32),
            pltpu.VMEM((zr, 128), jnp.float32),
            pltpu.SemaphoreType.DMA((NIDX + 2 * NBUF,)),
            pltpu.VMEM_SHARED((npad, 128), jnp.float32),
        ],
    )


def _tc_first_layer(x_all, t, wt, dis, n, d):
    def body(x_ref, w_ref, dis_ref, y_ref):
        x = x_ref[0]
        y = jnp.dot(x, w_ref[...], preferred_element_type=jnp.float32)
        y_ref[...] = y * dis_ref[...]

    f = x_all.shape[-1]
    return pl.pallas_call(
        body,
        grid=(1,),
        out_shape=jax.ShapeDtypeStruct((n, d), jnp.float32),
        in_specs=[
            pl.BlockSpec((1, n, f), lambda i: (t, 0, 0)),
            pl.BlockSpec((f, d), lambda i: (0, 0)),
            pl.BlockSpec((n, 1), lambda i: (0, 0)),
        ],
        out_specs=pl.BlockSpec((n, d), lambda i: (0, 0)),
    )(x_all, wt, dis)


def _tc_mid_layer(parts, wt, dis, b_prev, n, d):
    def body(a0_ref, a1_ref, w_ref, dis_ref, b_ref, y_ref):
        h = a0_ref[0] + a1_ref[0]
        h = h * dis_ref[...] + b_ref[...]
        h = h * jax.nn.sigmoid(h)
        y = jnp.dot(h, w_ref[...], preferred_element_type=jnp.float32)
        y_ref[...] = y * dis_ref[...]

    return pl.pallas_call(
        body,
        grid=(1,),
        out_shape=jax.ShapeDtypeStruct((n, d), jnp.float32),
        in_specs=[
            pl.BlockSpec((1, n, d), lambda i: (0, 0, 0)),
            pl.BlockSpec((1, n, d), lambda i: (1, 0, 0)),
            pl.BlockSpec((d, d), lambda i: (0, 0)),
            pl.BlockSpec((n, 1), lambda i: (0, 0)),
            pl.BlockSpec((1, d), lambda i: (0, 0)),
        ],
        out_specs=pl.BlockSpec((n, d), lambda i: (0, 0)),
    )(parts, parts, wt, dis, b_prev)


def _tc_final_layer(parts, dis, b3, n, d):
    def body(a0_ref, a1_ref, dis_ref, b_ref, out_ref):
        h = a0_ref[0] + a1_ref[0]
        h = h * dis_ref[...] + b_ref[...]
        h = h * jax.nn.sigmoid(h)
        out_ref[...] = jnp.sum(h, axis=0, keepdims=True) * (1.0 / n)

    return pl.pallas_call(
        body,
        grid=(1,),
        out_shape=jax.ShapeDtypeStruct((1, d), jnp.float32),
        in_specs=[
            pl.BlockSpec((1, n, d), lambda i: (0, 0, 0)),
            pl.BlockSpec((1, n, d), lambda i: (1, 0, 0)),
            pl.BlockSpec((n, 1), lambda i: (0, 0)),
            pl.BlockSpec((1, d), lambda i: (0, 0)),
        ],
        out_specs=pl.BlockSpec((1, d), lambda i: (0, 0)),
    )(parts, parts, dis, b3)


def _tc_temporal(emb, wih0, whh0, b0, wih1, whh1, b1, hw1, hb1, hw2, hb2):
    B, T, D = emb.shape
    H = D

    def body(e_ref, wih0_ref, whh0_ref, b0_ref, wih1_ref, whh1_ref, b1_ref,
             hw1_ref, hb1_ref, hw2_ref, hb2_ref, out_ref):
        x = e_ref[...]
        h0 = jnp.zeros((B, H), jnp.float32)
        c0 = jnp.zeros((B, H), jnp.float32)
        h1 = jnp.zeros((B, H), jnp.float32)
        c1 = jnp.zeros((B, H), jnp.float32)

        def step(xt, h, c, wih_ref, whh_ref, b_ref):
            gates = (jnp.dot(xt, wih_ref[...], preferred_element_type=jnp.float32)
                     + jnp.dot(h, whh_ref[...], preferred_element_type=jnp.float32)
                     + b_ref[...])
            i = jax.nn.sigmoid(gates[:, 0:H])
            f = jax.nn.sigmoid(gates[:, H:2 * H])
            g = jnp.tanh(gates[:, 2 * H:3 * H])
            o = jax.nn.sigmoid(gates[:, 3 * H:4 * H])
            c = f * c + i * g
            h = o * jnp.tanh(c)
            return h, c

        for t in range(T):
            xt = x[:, t, :]
            h0, c0 = step(xt, h0, c0, wih0_ref, whh0_ref, b0_ref)
            h1, c1 = step(h0, h1, c1, wih1_ref, whh1_ref, b1_ref)

        hid = jnp.dot(h1, hw1_ref[...], preferred_element_type=jnp.float32) + hb1_ref[...]
        hid = hid * jax.nn.sigmoid(hid)
        out_ref[...] = jnp.dot(hid, hw2_ref[...],
                               preferred_element_type=jnp.float32) + hb2_ref[...]

    nout = hw2.shape[1]
    full = lambda shape: pl.BlockSpec(shape, lambda i: (0,) * len(shape))
    return pl.pallas_call(
        body,
        grid=(1,),
        out_shape=jax.ShapeDtypeStruct((B, nout), jnp.float32),
        in_specs=[
            full((B, T, D)),
            full(wih0.shape), full(whh0.shape), full(b0.shape),
            full(wih1.shape), full(whh1.shape), full(b1.shape),
            full(hw1.shape), full(hb1.shape), full(hw2.shape), full(hb2.shape),
        ],
        out_specs=full((B, nout)),
    )(emb, wih0, whh0, b0, wih1, whh1, b1, hw1, hb1, hw2, hb2)


def kernel(snapshot_sequence, edge_index, conv1_W, conv1_b, conv2_W, conv2_b,
           conv3_W, conv3_b, lstm_W_ih0, lstm_W_hh0, lstm_b_ih0, lstm_b_hh0,
           lstm_W_ih1, lstm_W_hh1, lstm_b_ih1, lstm_b_hh1, head_W1, head_b1,
           head_W2, head_b2):
    B, T, N, F = snapshot_sequence.shape
    D = conv1_W.shape[0]
    E = edge_index.shape[1]
    BT = B * T

    npad = N + PADROWS
    assert npad % NS == 0
    edges = E + N
    ch = -(-edges // (NW * K))
    ch += (-ch) % NIDX
    e_pad = NW * ch * K
    pad = e_pad - edges

    idt = edge_index.dtype
    sl = jnp.arange(N, dtype=idt)
    pad_src = jnp.arange(pad, dtype=idt) % N
    pad_dst = N + jnp.arange(pad, dtype=idt) % PADROWS
    src_p = jnp.concatenate([edge_index[0], sl, pad_src]).reshape(NW, ch, K)
    dst_p = jnp.concatenate([edge_index[1], sl, pad_dst]).reshape(NW, ch, K)
    idx_p = jnp.stack([src_p, dst_p], axis=2)

    degp = _sc_degree_kernel(npad, ch)(idx_p)
    deg = degp[0] + degp[1]
    dis = jnp.where(deg > 0, lax.rsqrt(deg), 0.0)[:, None]

    w1t = conv1_W.T
    w2t = conv2_W.T
    w3t = conv3_W.T
    b1r = conv1_b[None, :]
    b2r = conv2_b[None, :]
    b3r = conv3_b[None, :]

    x_all = snapshot_sequence.reshape(BT, N, F)
    agg = _sc_aggregate_kernel(N, npad, ch)

    embs = []
    for t in range(BT):
        y1 = _tc_first_layer(x_all, t, w1t, dis, N, D)
        p1 = agg(y1, idx_p)
        y2 = _tc_mid_layer(p1, w2t, dis, b1r, N, D)
        p2 = agg(y2, idx_p)
        y3 = _tc_mid_layer(p2, w3t, dis, b2r, N, D)
        p3 = agg(y3, idx_p)
        embs.append(_tc_final_layer(p3, dis, b3r, N, D))

    emb = jnp.concatenate(embs, axis=0).reshape(B, T, D)

    out = _tc_temporal(
        emb,
        lstm_W_ih0.T, lstm_W_hh0.T, (lstm_b_ih0 + lstm_b_hh0)[None, :],
        lstm_W_ih1.T, lstm_W_hh1.T, (lstm_b_ih1 + lstm_b_hh1)[None, :],
        head_W1.T, head_b1[None, :], head_W2.T, head_b2[None, :])
    return out
